# baseline (speedup 1.0000x reference)
import functools

import jax
import jax.numpy as jnp
from jax import lax
from jax.experimental import pallas as pl
from jax.experimental.pallas import tpu as pltpu
from jax.experimental.pallas import tpu_sc as plsc

N = 2048
D = 1024
F = 4096
E = 8
K = 2
TILE = 256
NA = N * K
R_MAX = NA + E * TILE
NT = R_MAX // TILE
FFB = 256
NTPAD = 32
LANES = 16


def _router_body(x_ref, wg_ref, e_ref, w_ref, aux_ref):
    x = x_ref[...]
    wg = wg_ref[...]
    logits = jnp.dot(x, wg, preferred_element_type=jnp.float32)
    m = jnp.max(logits, axis=1, keepdims=True)
    ex = jnp.exp(logits - m)
    se = jnp.sum(ex, axis=1, keepdims=True)
    probs = ex / se
    iota = lax.broadcasted_iota(jnp.int32, (N, E), 1)
    p1 = jnp.max(probs, axis=1, keepdims=True)
    e1 = jnp.min(jnp.where(probs == p1, iota, E), axis=1, keepdims=True)
    probs2 = jnp.where(iota == e1, -1.0, probs)
    p2 = jnp.max(probs2, axis=1, keepdims=True)
    e2 = jnp.min(jnp.where(probs2 == p2, iota, E), axis=1, keepdims=True)
    s = p1 + p2 + 1e-6
    w_ref[:, 0:1] = p1 / s
    w_ref[:, 1:2] = p2 / s
    e_ref[:, 0:1] = e1
    e_ref[:, 1:2] = e2
    onehot = (iota == e1).astype(jnp.float32) + (iota == e2).astype(jnp.float32)
    tpe = jnp.sum(onehot, axis=0, keepdims=True) / (N * K)
    ppe = jnp.mean(probs, axis=0, keepdims=True)
    lb = E * jnp.sum(tpe * ppe)
    lse = m + jnp.log(se)
    z = jnp.mean(lse * lse) * 0.001
    aux_ref[...] = (0.01 * (lb + z)).reshape(1, 1)


def _router(xf, wg_t):
    return pl.pallas_call(
        _router_body,
        out_shape=[
            jax.ShapeDtypeStruct((N, K), jnp.int32),
            jax.ShapeDtypeStruct((N, K), jnp.float32),
            jax.ShapeDtypeStruct((1, 1), jnp.float32),
        ],
    )(xf, wg_t)


@functools.cache
def _sc_mesh():
    return plsc.VectorSubcoreMesh(core_axis_name="c", subcore_axis_name="s")


def _dispatch_body(e_hbm, w_hbm, src_hbm, wgt_hbm, pos_hbm, te_hbm, tv_hbm,
                   e_v, w_v, rank_v, src_v, wgt_v, pos_v, meta_v, counts_s):
    wid = lax.axis_index("s") * 2 + lax.axis_index("c")

    @pl.when(wid == 0)
    def _():
        pltpu.sync_copy(e_hbm, e_v)
        pltpu.sync_copy(w_hbm, w_v)
        for eid in range(E):
            counts_s[eid] = 0

        zi = jnp.zeros((LANES,), jnp.int32)
        zf = jnp.zeros((LANES,), jnp.float32)
        lane0 = lax.iota(jnp.int32, LANES)

        def _init(i, _):
            src_v[pl.ds(i * LANES, LANES)] = (i * LANES + lane0) & (N - 1)
            wgt_v[pl.ds(i * LANES, LANES)] = zf
            return 0
        lax.fori_loop(0, R_MAX // LANES, _init, 0)

        def _pass1(i, _):
            ee = e_v[pl.ds(i * LANES, LANES)]
            pos_c = zi
            for eid in range(E):
                mk = ee == eid
                mi = mk.astype(jnp.int32)
                cs = jnp.cumsum(mi)
                base = counts_s[eid]
                pos_c = jnp.where(mk, base + cs - 1, pos_c)
                counts_s[eid] = base + jnp.sum(mi)
            rank_v[pl.ds(i * LANES, LANES)] = pos_c
            return 0
        lax.fori_loop(0, NA // LANES, _pass1, 0)

        counts = [counts_s[eid] for eid in range(E)]
        offs = []
        acc = jnp.int32(0)
        for eid in range(E):
            offs.append(acc)
            acc = acc + ((counts[eid] + (TILE - 1)) & ~(TILE - 1))

        lane = lax.iota(jnp.int32, LANES)

        def _pass2(i, _):
            ee = e_v[pl.ds(i * LANES, LANES)]
            rr = rank_v[pl.ds(i * LANES, LANES)]
            ww = w_v[pl.ds(i * LANES, LANES)]
            base = zi
            for eid in range(E):
                base = jnp.where(ee == eid, offs[eid], base)
            p = base + rr
            a = i * LANES + lane
            tok = a >> 1
            kk = a & 1
            plsc.store_scatter(src_v, [p], tok)
            plsc.store_scatter(wgt_v, [p], ww)
            plsc.store_scatter(pos_v, [kk * N + tok], p)
            return 0
        lax.fori_loop(0, NA // LANES, _pass2, 0)

        for half in range(NTPAD // LANES):
            tstart = (lane + half * LANES) * TILE
            te = jnp.zeros((LANES,), jnp.int32) - 1
            for eid in range(E):
                te = te + jnp.where(offs[eid] <= tstart, 1, 0)
            te = jnp.clip(te, 0, E - 1)
            off_of = zi
            cnt_of = zi
            for eid in range(E):
                sel = te == eid
                off_of = jnp.where(sel, offs[eid], off_of)
                cnt_of = jnp.where(sel, counts[eid], cnt_of)
            tv = jnp.clip(cnt_of - (tstart - off_of), 0, TILE)
            meta_v[pl.ds(half * LANES, LANES)] = te
            meta_v[pl.ds(NTPAD + half * LANES, LANES)] = tv

        pltpu.sync_copy(src_v, src_hbm)
        pltpu.sync_copy(wgt_v, wgt_hbm)
        pltpu.sync_copy(pos_v, pos_hbm)
        pltpu.sync_copy(meta_v.at[pl.ds(0, NTPAD)], te_hbm)
        pltpu.sync_copy(meta_v.at[pl.ds(NTPAD, NTPAD)], tv_hbm)


def _dispatch(e_flat, w_flat):
    return pl.kernel(
        _dispatch_body,
        out_type=[
            jax.ShapeDtypeStruct((R_MAX,), jnp.int32),
            jax.ShapeDtypeStruct((R_MAX,), jnp.float32),
            jax.ShapeDtypeStruct((NA,), jnp.int32),
            jax.ShapeDtypeStruct((NTPAD,), jnp.int32),
            jax.ShapeDtypeStruct((NTPAD,), jnp.int32),
        ],
        mesh=_sc_mesh(),
        scratch_types=[
            pltpu.VMEM((NA,), jnp.int32),
            pltpu.VMEM((NA,), jnp.float32),
            pltpu.VMEM((NA,), jnp.int32),
            pltpu.VMEM((R_MAX,), jnp.int32),
            pltpu.VMEM((R_MAX,), jnp.float32),
            pltpu.VMEM((NA,), jnp.int32),
            pltpu.VMEM((2 * NTPAD,), jnp.int32),
            pltpu.SMEM((E,), jnp.int32),
        ],
        compiler_params=pltpu.CompilerParams(needs_layout_passes=False),
    )(e_flat, w_flat)


def _make_gather(nrows, chunk):
    nworkers = 32
    per_w = nrows // nworkers
    nch = per_w // chunk
    assert per_w % chunk == 0

    def body(table_hbm, idx_hbm, out_hbm, idx_v, buf_v, sem):
        wid = lax.axis_index("s") * 2 + lax.axis_index("c")
        for ch in range(nch):
            base = wid * per_w + ch * chunk
            pltpu.sync_copy(idx_hbm.at[pl.ds(base, chunk)], idx_v)
            pltpu.async_copy(table_hbm.at[idx_v], buf_v, sem).wait()
            pltpu.sync_copy(buf_v, out_hbm.at[pl.ds(base, chunk)])

    def run(table, idx):
        return pl.kernel(
            body,
            out_type=jax.ShapeDtypeStruct((nrows, D), jnp.float32),
            mesh=_sc_mesh(),
            scratch_types=[
                pltpu.VMEM((chunk,), jnp.int32),
                pltpu.VMEM((chunk, D), jnp.float32),
                pltpu.SemaphoreType.DMA,
            ],
        )(table, idx)

    return run


_gather_xs = _make_gather(R_MAX, 64)
_gather_out = _make_gather(NA, 64)


def _ffn_body(te_ref, tv_ref, xs_ref, g_ref, u_ref, d_ref, w_ref, o_ref):
    j = pl.program_id(0)
    t = pl.program_id(1)

    @pl.when(tv_ref[t] > 0)
    def _():
        x = xs_ref[pl.ds(t * TILE, TILE), :]
        g = g_ref[0]
        u = u_ref[0]
        dn = d_ref[0]
        hg = lax.dot_general(x, g, (((1,), (1,)), ((), ())),
                             preferred_element_type=jnp.float32)
        hu = lax.dot_general(x, u, (((1,), (1,)), ((), ())),
                             preferred_element_type=jnp.float32)
        h = (hg * lax.logistic(hg)) * hu * w_ref[0]
        part = lax.dot_general(h, dn, (((1,), (1,)), ((), ())),
                               preferred_element_type=jnp.float32)
        rows = pl.ds(t * TILE, TILE)

        @pl.when(j == 0)
        def _():
            o_ref[rows, :] = part

        @pl.when(j > 0)
        def _():
            o_ref[rows, :] += part


def _ffn(te, tv, xs, wgt3, gate_w, up_w, down_w):
    grid_spec = pltpu.PrefetchScalarGridSpec(
        num_scalar_prefetch=2,
        grid=(F // FFB, NT),
        in_specs=[
            pl.BlockSpec((R_MAX, D), lambda j, t, te, tv: (0, 0)),
            pl.BlockSpec((1, FFB, D), lambda j, t, te, tv: (te[t], j, 0)),
            pl.BlockSpec((1, FFB, D), lambda j, t, te, tv: (te[t], j, 0)),
            pl.BlockSpec((1, D, FFB), lambda j, t, te, tv: (te[t], 0, j)),
            pl.BlockSpec((1, TILE, 1), lambda j, t, te, tv: (t, 0, 0)),
        ],
        out_specs=pl.BlockSpec((R_MAX, D), lambda j, t, te, tv: (0, 0)),
    )
    return pl.pallas_call(
        _ffn_body,
        grid_spec=grid_spec,
        out_shape=jax.ShapeDtypeStruct((R_MAX, D), jnp.float32),
        compiler_params=pltpu.CompilerParams(
            dimension_semantics=("arbitrary", "arbitrary")),
    )(te, tv, xs, gate_w, up_w, down_w, wgt3)


_CB = 256


def _combine_body(x_ref, o0_ref, o1_ref, g_ref, b_ref, out_ref):
    y = x_ref[...] + o0_ref[...] + o1_ref[...]
    mu = jnp.mean(y, axis=1, keepdims=True)
    dlt = y - mu
    var = jnp.mean(dlt * dlt, axis=1, keepdims=True)
    yn = dlt * lax.rsqrt(var + 1e-5)
    out_ref[...] = yn * g_ref[...] + b_ref[...]


def _combine(xf, o0, o1, gamma, beta):
    bs = lambda: pl.BlockSpec((_CB, D), lambda i: (i, 0))
    return pl.pallas_call(
        _combine_body,
        grid=(N // _CB,),
        in_specs=[bs(), bs(), bs(),
                  pl.BlockSpec((1, D), lambda i: (0, 0)),
                  pl.BlockSpec((1, D), lambda i: (0, 0))],
        out_specs=bs(),
        out_shape=jax.ShapeDtypeStruct((N, D), jnp.float32),
    )(xf, o0, o1, gamma, beta)


def kernel(x, Wg, gate_w, up_w, down_w, ln_gamma, ln_beta):
    xf = x.reshape(N, D)
    ew, ww, aux = _router(xf, Wg.T)
    e_flat = ew.reshape(NA)
    w_flat = ww.reshape(NA)
    src, wgt, pos, te, tv = _dispatch(e_flat, w_flat)
    xs = _gather_xs(xf, src)
    ffn_out = _ffn(te, tv, xs, wgt.reshape(NT, TILE, 1),
                   gate_w, up_w, down_w)
    o_rows = _gather_out(ffn_out, pos)
    out = _combine(xf, o_rows[:N], o_rows[N:],
                   ln_gamma.reshape(1, D), ln_beta.reshape(1, D))
    return out.reshape(x.shape), aux[0, 0]


# speedup vs baseline: 1.4880x; 1.4880x over previous
import functools

import jax
import jax.numpy as jnp
from jax import lax
from jax.experimental import pallas as pl
from jax.experimental.pallas import tpu as pltpu
from jax.experimental.pallas import tpu_sc as plsc

N = 2048
D = 1024
F = 4096
E = 8
K = 2
TILE = 256
NA = N * K
R_MAX = NA + E * TILE
NT = R_MAX // TILE
FFB = 1024
NTPAD = 32
LANES = 16


def _router_body(x_ref, wg_ref, e_ref, w_ref, aux_ref):
    x = x_ref[...]
    wg = wg_ref[...]
    logits = jnp.dot(x, wg, preferred_element_type=jnp.float32)
    m = jnp.max(logits, axis=1, keepdims=True)
    ex = jnp.exp(logits - m)
    se = jnp.sum(ex, axis=1, keepdims=True)
    probs = ex / se
    iota = lax.broadcasted_iota(jnp.int32, (N, E), 1)
    p1 = jnp.max(probs, axis=1, keepdims=True)
    e1 = jnp.min(jnp.where(probs == p1, iota, E), axis=1, keepdims=True)
    probs2 = jnp.where(iota == e1, -1.0, probs)
    p2 = jnp.max(probs2, axis=1, keepdims=True)
    e2 = jnp.min(jnp.where(probs2 == p2, iota, E), axis=1, keepdims=True)
    s = p1 + p2 + 1e-6
    w_ref[:, 0:1] = p1 / s
    w_ref[:, 1:2] = p2 / s
    e_ref[:, 0:1] = e1
    e_ref[:, 1:2] = e2
    onehot = (iota == e1).astype(jnp.float32) + (iota == e2).astype(jnp.float32)
    tpe = jnp.sum(onehot, axis=0, keepdims=True) / (N * K)
    ppe = jnp.mean(probs, axis=0, keepdims=True)
    lb = E * jnp.sum(tpe * ppe)
    lse = m + jnp.log(se)
    z = jnp.mean(lse * lse) * 0.001
    aux_ref[...] = (0.01 * (lb + z)).reshape(1, 1)


def _router(xf, wg_t):
    return pl.pallas_call(
        _router_body,
        out_shape=[
            jax.ShapeDtypeStruct((N, K), jnp.int32),
            jax.ShapeDtypeStruct((N, K), jnp.float32),
            jax.ShapeDtypeStruct((1, 1), jnp.float32),
        ],
    )(xf, wg_t)


@functools.cache
def _sc_mesh():
    return plsc.VectorSubcoreMesh(core_axis_name="c", subcore_axis_name="s")


def _dispatch_body(e_hbm, w_hbm, src_hbm, wgt_hbm, pos_hbm, te_hbm, tv_hbm,
                   e_v, w_v, rank_v, src_v, wgt_v, pos_v, meta_v, counts_s):
    wid = lax.axis_index("s") * 2 + lax.axis_index("c")

    @pl.when(wid == 0)
    def _():
        pltpu.sync_copy(e_hbm, e_v)
        pltpu.sync_copy(w_hbm, w_v)
        for eid in range(E):
            counts_s[eid] = 0

        zi = jnp.zeros((LANES,), jnp.int32)
        zf = jnp.zeros((LANES,), jnp.float32)
        lane0 = lax.iota(jnp.int32, LANES)

        def _init(i, _):
            src_v[pl.ds(i * LANES, LANES)] = (i * LANES + lane0) & (N - 1)
            wgt_v[pl.ds(i * LANES, LANES)] = zf
            return 0
        lax.fori_loop(0, R_MAX // LANES, _init, 0)

        def _pass1(i, _):
            ee = e_v[pl.ds(i * LANES, LANES)]
            pos_c = zi
            for eid in range(E):
                mk = ee == eid
                mi = mk.astype(jnp.int32)
                cs = jnp.cumsum(mi)
                base = counts_s[eid]
                pos_c = jnp.where(mk, base + cs - 1, pos_c)
                counts_s[eid] = base + jnp.sum(mi)
            rank_v[pl.ds(i * LANES, LANES)] = pos_c
            return 0
        lax.fori_loop(0, NA // LANES, _pass1, 0)

        counts = [counts_s[eid] for eid in range(E)]
        offs = []
        acc = jnp.int32(0)
        for eid in range(E):
            offs.append(acc)
            acc = acc + ((counts[eid] + (TILE - 1)) & ~(TILE - 1))

        lane = lax.iota(jnp.int32, LANES)

        def _pass2(i, _):
            ee = e_v[pl.ds(i * LANES, LANES)]
            rr = rank_v[pl.ds(i * LANES, LANES)]
            ww = w_v[pl.ds(i * LANES, LANES)]
            base = zi
            for eid in range(E):
                base = jnp.where(ee == eid, offs[eid], base)
            p = base + rr
            a = i * LANES + lane
            tok = a >> 1
            kk = a & 1
            plsc.store_scatter(src_v, [p], tok)
            plsc.store_scatter(wgt_v, [p], ww)
            plsc.store_scatter(pos_v, [kk * N + tok], p)
            return 0
        lax.fori_loop(0, NA // LANES, _pass2, 0)

        for half in range(NTPAD // LANES):
            tstart = (lane + half * LANES) * TILE
            te = jnp.zeros((LANES,), jnp.int32) - 1
            for eid in range(E):
                te = te + jnp.where(offs[eid] <= tstart, 1, 0)
            te = jnp.clip(te, 0, E - 1)
            off_of = zi
            cnt_of = zi
            for eid in range(E):
                sel = te == eid
                off_of = jnp.where(sel, offs[eid], off_of)
                cnt_of = jnp.where(sel, counts[eid], cnt_of)
            tv = jnp.clip(cnt_of - (tstart - off_of), 0, TILE)
            meta_v[pl.ds(half * LANES, LANES)] = te
            meta_v[pl.ds(NTPAD + half * LANES, LANES)] = tv

        pltpu.sync_copy(src_v, src_hbm)
        pltpu.sync_copy(wgt_v, wgt_hbm)
        pltpu.sync_copy(pos_v, pos_hbm)
        pltpu.sync_copy(meta_v.at[pl.ds(0, NTPAD)], te_hbm)
        pltpu.sync_copy(meta_v.at[pl.ds(NTPAD, NTPAD)], tv_hbm)


def _dispatch(e_flat, w_flat):
    return pl.kernel(
        _dispatch_body,
        out_type=[
            jax.ShapeDtypeStruct((R_MAX,), jnp.int32),
            jax.ShapeDtypeStruct((R_MAX,), jnp.float32),
            jax.ShapeDtypeStruct((NA,), jnp.int32),
            jax.ShapeDtypeStruct((NTPAD,), jnp.int32),
            jax.ShapeDtypeStruct((NTPAD,), jnp.int32),
        ],
        mesh=_sc_mesh(),
        scratch_types=[
            pltpu.VMEM((NA,), jnp.int32),
            pltpu.VMEM((NA,), jnp.float32),
            pltpu.VMEM((NA,), jnp.int32),
            pltpu.VMEM((R_MAX,), jnp.int32),
            pltpu.VMEM((R_MAX,), jnp.float32),
            pltpu.VMEM((NA,), jnp.int32),
            pltpu.VMEM((2 * NTPAD,), jnp.int32),
            pltpu.SMEM((E,), jnp.int32),
        ],
        compiler_params=pltpu.CompilerParams(needs_layout_passes=False),
    )(e_flat, w_flat)


def _make_gather(nrows, chunk):
    nworkers = 32
    per_w = nrows // nworkers
    nch = per_w // chunk
    assert per_w % chunk == 0

    def body(table_hbm, idx_hbm, out_hbm, idx_v, buf_v, sem):
        wid = lax.axis_index("s") * 2 + lax.axis_index("c")
        for ch in range(nch):
            base = wid * per_w + ch * chunk
            pltpu.sync_copy(idx_hbm.at[pl.ds(base, chunk)], idx_v)
            pltpu.async_copy(table_hbm.at[idx_v], buf_v, sem).wait()
            pltpu.sync_copy(buf_v, out_hbm.at[pl.ds(base, chunk)])

    def run(table, idx):
        return pl.kernel(
            body,
            out_type=jax.ShapeDtypeStruct((nrows, D), jnp.float32),
            mesh=_sc_mesh(),
            scratch_types=[
                pltpu.VMEM((chunk,), jnp.int32),
                pltpu.VMEM((chunk, D), jnp.float32),
                pltpu.SemaphoreType.DMA,
            ],
        )(table, idx)

    return run


_gather_xs = _make_gather(R_MAX, 64)
_gather_out = _make_gather(NA, 64)


def _ffn_body(te_ref, tv_ref, xs_ref, g_ref, u_ref, d_ref, w_ref, o_ref):
    j = pl.program_id(0)
    t = pl.program_id(1)

    @pl.when(tv_ref[t] > 0)
    def _():
        x = xs_ref[...]
        g = g_ref[0]
        u = u_ref[0]
        dn = d_ref[0]
        hg = lax.dot_general(x, g, (((1,), (1,)), ((), ())),
                             preferred_element_type=jnp.float32)
        hu = lax.dot_general(x, u, (((1,), (1,)), ((), ())),
                             preferred_element_type=jnp.float32)
        h = (hg * lax.logistic(hg)) * hu * w_ref[0]
        part = lax.dot_general(h, dn, (((1,), (1,)), ((), ())),
                               preferred_element_type=jnp.float32)
        rows = pl.ds(t * TILE, TILE)

        @pl.when(j == 0)
        def _():
            o_ref[rows, :] = part

        @pl.when(j > 0)
        def _():
            o_ref[rows, :] += part


def _ffn(te, tv, xs, wgt3, gate_w, up_w, down_w):
    grid_spec = pltpu.PrefetchScalarGridSpec(
        num_scalar_prefetch=2,
        grid=(F // FFB, NT),
        in_specs=[
            pl.BlockSpec((TILE, D), lambda j, t, te, tv: (t, 0)),
            pl.BlockSpec((1, FFB, D), lambda j, t, te, tv: (te[t], j, 0)),
            pl.BlockSpec((1, FFB, D), lambda j, t, te, tv: (te[t], j, 0)),
            pl.BlockSpec((1, D, FFB), lambda j, t, te, tv: (te[t], 0, j)),
            pl.BlockSpec((1, TILE, 1), lambda j, t, te, tv: (t, 0, 0)),
        ],
        out_specs=pl.BlockSpec((R_MAX, D), lambda j, t, te, tv: (0, 0)),
    )
    return pl.pallas_call(
        _ffn_body,
        grid_spec=grid_spec,
        out_shape=jax.ShapeDtypeStruct((R_MAX, D), jnp.float32),
        compiler_params=pltpu.CompilerParams(
            dimension_semantics=("arbitrary", "arbitrary")),
    )(te, tv, xs, gate_w, up_w, down_w, wgt3)


_CB = 256


def _combine_body(x_ref, o0_ref, o1_ref, g_ref, b_ref, out_ref):
    y = x_ref[...] + o0_ref[...] + o1_ref[...]
    mu = jnp.mean(y, axis=1, keepdims=True)
    dlt = y - mu
    var = jnp.mean(dlt * dlt, axis=1, keepdims=True)
    yn = dlt * lax.rsqrt(var + 1e-5)
    out_ref[...] = yn * g_ref[...] + b_ref[...]


def _combine(xf, o0, o1, gamma, beta):
    bs = lambda: pl.BlockSpec((_CB, D), lambda i: (i, 0))
    return pl.pallas_call(
        _combine_body,
        grid=(N // _CB,),
        in_specs=[bs(), bs(), bs(),
                  pl.BlockSpec((1, D), lambda i: (0, 0)),
                  pl.BlockSpec((1, D), lambda i: (0, 0))],
        out_specs=bs(),
        out_shape=jax.ShapeDtypeStruct((N, D), jnp.float32),
    )(xf, o0, o1, gamma, beta)


def kernel(x, Wg, gate_w, up_w, down_w, ln_gamma, ln_beta):
    xf = x.reshape(N, D)
    ew, ww, aux = _router(xf, Wg.T)
    e_flat = ew.reshape(NA)
    w_flat = ww.reshape(NA)
    src, wgt, pos, te, tv = _dispatch(e_flat, w_flat)
    xs = _gather_xs(xf, src)
    ffn_out = _ffn(te, tv, xs, wgt.reshape(NT, TILE, 1),
                   gate_w, up_w, down_w)
    o_rows = _gather_out(ffn_out, pos)
    out = _combine(xf, o_rows[:N], o_rows[N:],
                   ln_gamma.reshape(1, D), ln_beta.reshape(1, D))
    return out.reshape(x.shape), aux[0, 0]
